# baseline (speedup 1.0000x reference)
import jax
import jax.numpy as jnp
from jax import lax
from jax.experimental import pallas as pl
from jax.experimental.pallas import tpu as pltpu
from jax.experimental.pallas import tpu_sc as plsc

VOCAB = 1000
EMB_DIM = 4
B, L = 4096, 50
LP = 56

NC, NS = 2, 16
NW = NC * NS
ROWS_W = B // NW


def _table_body(emb_ref, w_ref, b_ref, out_ref):
    acc = jax.lax.dot_general(
        emb_ref[...], w_ref[...],
        (((1,), (1,)), ((), ())),
        preferred_element_type=jnp.float32,
    )
    out_ref[...] = acc + b_ref[...]


def _make_table(emb_table, W, b):
    return pl.pallas_call(
        _table_body,
        out_shape=jax.ShapeDtypeStruct((VOCAB, VOCAB), jnp.float32),
    )(emb_table, W, b.reshape(1, VOCAB))


def _sc_gather_body(table_hbm, x_hbm, out_hbm, idx_v, rows0, rows1, sem0, sem1):
    c = lax.axis_index("c")
    s = lax.axis_index("s")
    wid = s * NC + c
    row_base = wid * ROWS_W

    pltpu.sync_copy(x_hbm.at[pl.ds(wid * (ROWS_W * LP), ROWS_W * LP)], idx_v)

    def gather(g, rows_v, sem):
        src = table_hbm.at[idx_v.at[pl.ds(g * LP, L)]]
        return pltpu.make_async_copy(src, rows_v.at[0], sem)

    def emit(g, rows_v):
        pltpu.sync_copy(rows_v, out_hbm.at[pl.ds(row_base + g, 1)])

    gather(0, rows0, sem0).start()

    def pair(i, carry):
        g0 = i * 2
        gather(g0 + 1, rows1, sem1).start()
        gather(g0, rows0, sem0).wait()
        emit(g0, rows0)

        @pl.when(g0 + 2 < ROWS_W)
        def _():
            gather(g0 + 2, rows0, sem0).start()

        gather(g0 + 1, rows1, sem1).wait()
        emit(g0 + 1, rows1)
        return carry

    lax.fori_loop(0, ROWS_W // 2, pair, 0)


def _gather_rows(table, x_flat):
    mesh = plsc.VectorSubcoreMesh(core_axis_name="c", subcore_axis_name="s")
    return pl.kernel(
        _sc_gather_body,
        out_type=jax.ShapeDtypeStruct((B, L, VOCAB), jnp.float32),
        mesh=mesh,
        scratch_types=[
            pltpu.VMEM((ROWS_W * LP,), jnp.int32),
            pltpu.VMEM((1, L, VOCAB), jnp.float32),
            pltpu.VMEM((1, L, VOCAB), jnp.float32),
            pltpu.SemaphoreType.DMA,
            pltpu.SemaphoreType.DMA,
        ],
        compiler_params=pltpu.CompilerParams(use_tc_tiling_on_sc=False),
    )(table, x_flat)


def kernel(x, emb_table, W, b):
    table = _make_table(emb_table, W, b)
    x_pad = jnp.pad(x.astype(jnp.int32), ((0, 0), (0, LP - L)))
    return _gather_rows(table, x_pad.reshape(-1))


# speedup vs baseline: 1.0754x; 1.0754x over previous
import jax
import jax.numpy as jnp
from jax import lax
from jax.experimental import pallas as pl
from jax.experimental.pallas import tpu as pltpu
from jax.experimental.pallas import tpu_sc as plsc

VOCAB = 1000
EMB_DIM = 4
B, L = 4096, 50

NC, NS = 2, 16
NW = NC * NS
BT = B // NW
VT = VOCAB // 8
QW = 32
NQ = BT // QW
LANES = 16


def _table_body(emb_ref, w_ref, b_ref, out_ref):
    acc = jax.lax.dot_general(
        emb_ref[...], w_ref[...],
        (((1,), (1,)), ((), ())),
        preferred_element_type=jnp.float32,
    )
    out_ref[...] = acc + b_ref[...]


def _make_table(emb_table, W, b):
    return pl.pallas_call(
        _table_body,
        out_shape=jax.ShapeDtypeStruct((VOCAB, VOCAB), jnp.float32),
    )(emb_table, W, b.reshape(1, VOCAB))


def _sc_gather_body(table_hbm, xt_hbm, out_hbm,
                    idx_v, rows0, rows1, tr0, tr1,
                    gs0, gs1, ws0, ws1):
    c = lax.axis_index("c")
    s = lax.axis_index("s")
    wid = s * NC + c

    iota = lax.iota(jnp.int32, LANES)

    def gather(q, rows_v, sem):
        src = table_hbm.at[idx_v.at[pl.ds(q * QW, QW)]]
        return pltpu.make_async_copy(src, rows_v, sem)

    def out_dma(l, q, tr_v, sem):
        dst = out_hbm.at[l, pl.ds(0, VT), wid, pl.ds(0, 8), pl.ds(q * QW, QW)]
        return pltpu.make_async_copy(tr_v, dst, sem)

    def transpose(rows_v, tr_v):
        def vt_body(vt, carry):
            for vi in range(8):
                col = vt * 8 + vi
                col_v = jnp.full((LANES,), col, jnp.int32)
                for b2 in range(0, QW, LANES):
                    vals = plsc.load_gather(rows_v, [b2 + iota, col_v])
                    tr_v[vt, vi, pl.ds(b2, LANES)] = vals
            return carry
        lax.fori_loop(0, VT, vt_body, 0)

    def l_body(l, carry):
        pltpu.sync_copy(xt_hbm.at[l, pl.ds(wid * BT, BT)], idx_v)
        gather(0, rows0, gs0).start()
        gather(1, rows1, gs1).start()
        for q in range(NQ):
            rows_v, gsem = (rows0, gs0) if q % 2 == 0 else (rows1, gs1)
            tr_v, wsem = (tr0, ws0) if q % 2 == 0 else (tr1, ws1)
            gather(q, rows_v, gsem).wait()

            @pl.when(l * NQ + q >= 2)
            def _():
                out_dma(l, q, tr_v, wsem).wait()

            transpose(rows_v, tr_v)
            out_dma(l, q, tr_v, wsem).start()
            if q + 2 < NQ:
                gather(q + 2, rows_v, gsem).start()
        return carry

    lax.fori_loop(0, L, l_body, 0)
    out_dma(L - 1, NQ - 2, tr0, ws0).wait()
    out_dma(L - 1, NQ - 1, tr1, ws1).wait()


def _gather_rows(table, x_t):
    mesh = plsc.VectorSubcoreMesh(core_axis_name="c", subcore_axis_name="s")
    return pl.kernel(
        _sc_gather_body,
        out_type=jax.ShapeDtypeStruct((L, VT, NW, 8, BT), jnp.float32),
        mesh=mesh,
        scratch_types=[
            pltpu.VMEM((BT,), jnp.int32),
            pltpu.VMEM((QW, VOCAB), jnp.float32),
            pltpu.VMEM((QW, VOCAB), jnp.float32),
            pltpu.VMEM((VT, 8, QW), jnp.float32),
            pltpu.VMEM((VT, 8, QW), jnp.float32),
            pltpu.SemaphoreType.DMA,
            pltpu.SemaphoreType.DMA,
            pltpu.SemaphoreType.DMA,
            pltpu.SemaphoreType.DMA,
        ],
        compiler_params=pltpu.CompilerParams(
            use_tc_tiling_on_sc=False, needs_layout_passes=False
        ),
    )(table, x_t)


def kernel(x, emb_table, W, b):
    table = _make_table(emb_table, W, b)
    x_t = x.astype(jnp.int32).T
    out5 = _gather_rows(table, x_t)
    return out5.transpose(2, 4, 0, 1, 3).reshape(B, L, VOCAB)


# speedup vs baseline: 3.0613x; 2.8468x over previous
import jax
import jax.numpy as jnp
from jax import lax
from jax.experimental import pallas as pl
from jax.experimental.pallas import tpu as pltpu
from jax.experimental.pallas import tpu_sc as plsc

VOCAB = 1000
EMB_DIM = 4
B, L = 4096, 50

NC, NS = 2, 16
NW = NC * NS
BT = B // NW
VT = VOCAB // 8
QW = 16
NQ = BT // QW
LANES = 16


def _table_body(emb_ref, w_ref, b_ref, out_ref):
    acc = jax.lax.dot_general(
        emb_ref[...], w_ref[...],
        (((1,), (1,)), ((), ())),
        preferred_element_type=jnp.float32,
    )
    out_ref[...] = acc + b_ref[...]


def _make_table(emb_table, W, b):
    return pl.pallas_call(
        _table_body,
        out_shape=jax.ShapeDtypeStruct((VOCAB, VOCAB), jnp.float32),
    )(emb_table, W, b.reshape(1, VOCAB))


def _sc_gather_body(table_hbm, xt_hbm, out_hbm,
                    idx_v, rows0, rows1, tr0, tr1,
                    gs0, gs1, ws0, ws1):
    c = lax.axis_index("c")
    s = lax.axis_index("s")
    wid = s * NC + c

    iota = lax.iota(jnp.int32, LANES)
    bvecs = [iota + b2 for b2 in range(0, QW, LANES)]

    pltpu.sync_copy(xt_hbm.at[pl.ds(0, L), pl.ds(wid * BT, BT)], idx_v)

    NQTOT = L * NQ

    def gather(Q, rows_v, sem):
        l = Q // NQ
        q = lax.rem(Q, NQ)
        src = table_hbm.at[idx_v.at[l, pl.ds(q * QW, QW)]]
        return pltpu.make_async_copy(src, rows_v, sem)

    def out_dma(Q, tr_v, sem):
        l = Q // NQ
        q = lax.rem(Q, NQ)
        dst = out_hbm.at[l, pl.ds(0, VT), wid, pl.ds(0, 8), pl.ds(q * QW, QW)]
        return pltpu.make_async_copy(tr_v, dst, sem)

    def transpose(rows_v, tr_v):
        @plsc.parallel_loop(0, VT, unroll=4)
        def _(vt):
            for vi in range(8):
                col_v = jnp.full((LANES,), vt * 8 + vi, jnp.int32)
                for k, bv in enumerate(bvecs):
                    tr_v[vt, vi, pl.ds(k * LANES, LANES)] = (
                        plsc.load_gather(rows_v, [bv, col_v]))

    gather(0, rows0, gs0).start()
    gather(1, rows1, gs1).start()

    def pair(t2, carry):
        for k in range(2):
            Q = t2 * 2 + k
            rows_v, gsem = (rows0, gs0) if k == 0 else (rows1, gs1)
            tr_v, wsem = (tr0, ws0) if k == 0 else (tr1, ws1)
            gather(Q, rows_v, gsem).wait()

            @pl.when(Q >= 2)
            def _():
                out_dma(Q, tr_v, wsem).wait()

            transpose(rows_v, tr_v)
            out_dma(Q, tr_v, wsem).start()

            @pl.when(Q + 2 < NQTOT)
            def _():
                gather(Q + 2, rows_v, gsem).start()
        return carry

    lax.fori_loop(0, NQTOT // 2, pair, 0)
    out_dma(NQTOT - 2, tr0, ws0).wait()
    out_dma(NQTOT - 1, tr1, ws1).wait()


def _gather_rows(table, x_t):
    mesh = plsc.VectorSubcoreMesh(core_axis_name="c", subcore_axis_name="s")
    return pl.kernel(
        _sc_gather_body,
        out_type=jax.ShapeDtypeStruct((L, VT, NW, 8, BT), jnp.float32),
        mesh=mesh,
        scratch_types=[
            pltpu.VMEM((L, BT), jnp.int32),
            pltpu.VMEM((QW, VOCAB), jnp.float32),
            pltpu.VMEM((QW, VOCAB), jnp.float32),
            pltpu.VMEM((VT, 8, QW), jnp.float32),
            pltpu.VMEM((VT, 8, QW), jnp.float32),
            pltpu.SemaphoreType.DMA,
            pltpu.SemaphoreType.DMA,
            pltpu.SemaphoreType.DMA,
            pltpu.SemaphoreType.DMA,
        ],
        compiler_params=pltpu.CompilerParams(
            use_tc_tiling_on_sc=False, needs_layout_passes=False
        ),
    )(table, x_t)


def kernel(x, emb_table, W, b):
    table = _make_table(emb_table, W, b)
    x_t = x.astype(jnp.int32).T
    out5 = _gather_rows(table, x_t)
    return out5.transpose(2, 4, 0, 1, 3).reshape(B, L, VOCAB)
